# baseline (device time: 13876 ns/iter reference)
import jax
import jax.numpy as jnp
from jax import lax
from jax.experimental import pallas as pl
from jax.experimental.pallas import tpu as pltpu

N_DEV = 4


def kernel(A, B):
    m_per, k = A.shape
    _, n = B.shape
    half = m_per // 2

    def body(a_ref, b_ref, out_ref, a_bf, from_l, from_r, from_opp,
             send_sems, recv_sems):
        my_pos = lax.axis_index("i")
        left = (my_pos - 1) % N_DEV
        right = (my_pos + 1) % N_DEV
        opp = (my_pos + 2) % N_DEV

        q = half // 2
        top = pl.ds(0, half)
        bot = pl.ds(half, half)
        q1 = pl.ds(0, q)
        q2 = pl.ds(q, q)
        q3 = pl.ds(half, q)
        q4 = pl.ds(half + q, q)

        def copy(src, dst, s_sem, r_sem, target):
            return pltpu.make_async_remote_copy(
                src_ref=src, dst_ref=dst,
                send_sem=send_sems.at[s_sem], recv_sem=recv_sems.at[r_sem],
                device_id=(target,), device_id_type=pl.DeviceIdType.MESH,
            )

        a_bf[...] = a_ref[...].astype(jnp.bfloat16)

        barrier_sem = pltpu.get_barrier_semaphore()
        for nbr in [left, right]:
            pl.semaphore_signal(
                barrier_sem, inc=1,
                device_id=(nbr,), device_id_type=pl.DeviceIdType.MESH,
            )
        pl.semaphore_wait(barrier_sem, 2)

        sr_q1 = copy(a_bf.at[q1], from_l.at[q1], 0, 0, right)
        sr_q2 = copy(a_bf.at[q2], from_l.at[q2], 1, 1, right)
        sr_bot = copy(a_bf.at[bot], from_l.at[bot], 2, 2, right)
        sl_q3 = copy(a_bf.at[q3], from_r.at[q3], 3, 3, left)
        sl_q4 = copy(a_bf.at[q4], from_r.at[q4], 4, 4, left)
        sl_top = copy(a_bf.at[top], from_r.at[top], 5, 5, left)
        sr_q1.start()
        sl_q3.start()
        sr_q2.start()
        sl_q4.start()
        sr_bot.start()
        sl_top.start()

        out_ref[pl.ds(my_pos * m_per, m_per), :] = jnp.dot(
            a_ref[...], b_ref[...], preferred_element_type=jnp.float32
        )

        relay_r1 = copy(from_l.at[q1], from_opp.at[q1], 6, 6, right)
        relay_r2 = copy(from_l.at[q2], from_opp.at[q2], 7, 7, right)
        relay_l1 = copy(from_r.at[q3], from_opp.at[q3], 8, 8, left)
        relay_l2 = copy(from_r.at[q4], from_opp.at[q4], 9, 9, left)
        sr_q1.wait_recv()
        relay_r1.start()
        sl_q3.wait_recv()
        relay_l1.start()
        sr_q2.wait_recv()
        relay_r2.start()
        sl_q4.wait_recv()
        relay_l2.start()

        sr_bot.wait_recv()
        out_ref[pl.ds(left * m_per, m_per), :] = jnp.dot(
            from_l[...].astype(jnp.float32), b_ref[...],
            preferred_element_type=jnp.float32,
        )
        sl_top.wait_recv()
        out_ref[pl.ds(right * m_per, m_per), :] = jnp.dot(
            from_r[...].astype(jnp.float32), b_ref[...],
            preferred_element_type=jnp.float32,
        )

        relay_r1.wait_recv()
        relay_r2.wait_recv()
        out_ref[pl.ds(opp * m_per, half), :] = jnp.dot(
            from_opp[:half, :].astype(jnp.float32), b_ref[...],
            preferred_element_type=jnp.float32,
        )
        relay_l1.wait_recv()
        relay_l2.wait_recv()
        out_ref[pl.ds(opp * m_per + half, half), :] = jnp.dot(
            from_opp[half:, :].astype(jnp.float32), b_ref[...],
            preferred_element_type=jnp.float32,
        )

        for r in [sr_q1, sr_q2, sr_bot, sl_q3, sl_q4, sl_top,
                  relay_r1, relay_r2, relay_l1, relay_l2]:
            r.wait_send()

    return pl.pallas_call(
        body,
        out_shape=jax.ShapeDtypeStruct((N_DEV * m_per, n), jnp.float32),
        in_specs=[
            pl.BlockSpec(memory_space=pltpu.VMEM),
            pl.BlockSpec(memory_space=pltpu.VMEM),
        ],
        out_specs=pl.BlockSpec(memory_space=pltpu.VMEM),
        scratch_shapes=[
            pltpu.VMEM((m_per, k), jnp.bfloat16),
            pltpu.VMEM((m_per, k), jnp.bfloat16),
            pltpu.VMEM((m_per, k), jnp.bfloat16),
            pltpu.VMEM((m_per, k), jnp.bfloat16),
            pltpu.SemaphoreType.DMA((10,)),
            pltpu.SemaphoreType.DMA((10,)),
        ],
        compiler_params=pltpu.CompilerParams(collective_id=0),
    )(A, B)
